# baseline (speedup 1.0000x reference)
import functools

import jax
import jax.numpy as jnp
from jax import lax
from jax.experimental import pallas as pl
from jax.experimental.pallas import tpu as pltpu
from jax.experimental.pallas import tpu_sc as plsc

_NC = 2
_NS = 16
_NW = _NC * _NS
_LANES = 16
_CH = 128


def _sc_body(nper, embed,
             row_id, col_id, row_emb, col_emb, row_bias, col_bias, gvec,
             out,
             ridx_v, cidx_v, re_v, ce_v, gb_v, out_v, sem):
    wid = lax.axis_index("s") * _NC + lax.axis_index("c")
    base = wid * nper
    nchunk = nper // _CH

    pltpu.sync_copy(row_id.at[pl.ds(base, nper)], ridx_v)
    pltpu.sync_copy(col_id.at[pl.ds(base, nper)], cidx_v)
    pltpu.sync_copy(gvec, gb_v)

    lanes = lax.iota(jnp.int32, _LANES)
    msk_last = lanes == (_LANES - 1)
    zeros = jnp.zeros((_LANES,), jnp.float32)
    gbm = jnp.where(lanes == 0, gb_v[...], zeros)
    nvec = embed // _LANES

    for ch in range(nchunk):
        off = ch * _CH

        def issue(g, carry):
            rv = ridx_v[pl.ds(off + g * _LANES, _LANES)]
            cv = cidx_v[pl.ds(off + g * _LANES, _LANES)]
            for i in range(_LANES):
                r = g * _LANES + i
                rid = rv[i]
                cid = cv[i]
                pltpu.make_async_copy(row_emb.at[rid], re_v.at[r, pl.ds(0, embed)], sem).start()
                pltpu.make_async_copy(col_emb.at[cid], ce_v.at[r, pl.ds(0, embed)], sem).start()
                pltpu.make_async_copy(row_bias.at[rid], re_v.at[r, pl.ds(embed, 1)], sem).start()
                pltpu.make_async_copy(col_bias.at[cid], ce_v.at[r, pl.ds(embed, 1)], sem).start()
            return carry

        lax.fori_loop(0, _CH // _LANES, issue, 0)

        def drain(g, carry):
            rv = ridx_v[pl.ds(off + g * _LANES, _LANES)]
            cv = cidx_v[pl.ds(off + g * _LANES, _LANES)]
            for i in range(_LANES):
                r = g * _LANES + i
                rid = rv[i]
                cid = cv[i]
                pltpu.make_async_copy(row_emb.at[rid], re_v.at[r, pl.ds(0, embed)], sem).wait()
                pltpu.make_async_copy(col_emb.at[cid], ce_v.at[r, pl.ds(0, embed)], sem).wait()
                pltpu.make_async_copy(row_bias.at[rid], re_v.at[r, pl.ds(embed, 1)], sem).wait()
                pltpu.make_async_copy(col_bias.at[cid], ce_v.at[r, pl.ds(embed, 1)], sem).wait()
            return carry

        lax.fori_loop(0, _CH // _LANES, drain, 0)

        def g_body(g, carry):
            for i in range(_LANES):
                r = g * _LANES + i
                v = re_v[r, pl.ds(0, _LANES)] * ce_v[r, pl.ds(0, _LANES)]
                for k in range(1, nvec):
                    v = v + re_v[r, pl.ds(k * _LANES, _LANES)] * ce_v[r, pl.ds(k * _LANES, _LANES)]
                rbv = re_v[r, pl.ds(embed, _LANES)]
                cbv = ce_v[r, pl.ds(embed, _LANES)]
                v = v + gbm + jnp.where(lanes == 0, rbv + cbv, zeros)
                c = plsc.cumsum(v)
                plsc.store_scatter(out_v, [jnp.full((_LANES,), off + r, jnp.int32)], c,
                                   mask=msk_last)
            return carry

        lax.fori_loop(0, _CH // _LANES, g_body, 0)

    pltpu.sync_copy(out_v, out.at[pl.ds(base, nper)])


def _sc_matfac(rid, cid, row_emb, col_emb, row_bias, col_bias, gvec):
    batch = rid.shape[0]
    embed = row_emb.shape[1]
    nper = batch // _NW
    assert nper * _NW == batch and nper % _CH == 0

    run = pl.kernel(
        functools.partial(_sc_body, nper, embed),
        out_type=jax.ShapeDtypeStruct((batch,), jnp.float32),
        mesh=plsc.VectorSubcoreMesh(core_axis_name="c", subcore_axis_name="s"),
        compiler_params=pltpu.CompilerParams(needs_layout_passes=False,
                                             use_tc_tiling_on_sc=True),
        scratch_types=[
            pltpu.VMEM((nper,), jnp.int32),
            pltpu.VMEM((nper,), jnp.int32),
            pltpu.VMEM((_CH, embed + _LANES), jnp.float32),
            pltpu.VMEM((_CH, embed + _LANES), jnp.float32),
            pltpu.VMEM((_LANES,), jnp.float32),
            pltpu.VMEM((nper,), jnp.float32),
            pltpu.SemaphoreType.DMA,
        ],
    )
    return run(rid, cid, row_emb, col_emb, row_bias, col_bias, gvec)


def kernel(row_id, col_id, row_emb, row_bias, col_emb, col_bias, global_bias):
    batch = row_id.shape[0]
    rid = row_id.astype(jnp.int32)
    cid = col_id.astype(jnp.int32)
    gvec = jnp.full((_LANES,), global_bias, jnp.float32)
    out = _sc_matfac(rid, cid, row_emb, col_emb, row_bias, col_bias, gvec)
    return out.reshape(batch, 1)


# speedup vs baseline: 2.1805x; 2.1805x over previous
import functools

import jax
import jax.numpy as jnp
from jax import lax
from jax.experimental import pallas as pl
from jax.experimental.pallas import tpu as pltpu
from jax.experimental.pallas import tpu_sc as plsc

_NC = 2
_NS = 16
_NW = _NC * _NS
_LANES = 16
_CH = 128


def _sc_body(nper, embed, vm, vt,
             row_id, col_id, dflat_r, dflat_c, tail_r, tail_c,
             row_bias, col_bias, gvec, out,
             ridx_v, cidx_v, ribuf, cibuf, re_v, ce_v, tr_v, tc_v,
             rb_v, cb_v, gb_v, out_v, sem):
    wid = lax.axis_index("s") * _NC + lax.axis_index("c")
    base = wid * nper
    nchunk = nper // _CH
    ngrp = _CH // _LANES
    tile_row_words = (vm // 128) * 1024

    for j in range(nchunk):
        pltpu.sync_copy(row_id.at[pl.ds(base + j * _CH, _CH)], ridx_v.at[j])
        pltpu.sync_copy(col_id.at[pl.ds(base + j * _CH, _CH)], cidx_v.at[j])
    if vt:
        pltpu.sync_copy(tail_r, tr_v)
        pltpu.sync_copy(tail_c, tc_v)
    pltpu.sync_copy(gvec, gb_v)
    gb = gb_v[...]

    for ch in range(nchunk):
        off = ch * _CH

        pltpu.make_async_copy(row_bias.at[ridx_v.at[ch]],
                              rb_v.at[pl.ds(off, _CH)], sem).start()
        pltpu.make_async_copy(col_bias.at[cidx_v.at[ch]],
                              cb_v.at[pl.ds(off, _CH)], sem).start()

        def build(g, carry):
            gsl = pl.ds(g * _LANES, _LANES)
            r16 = jnp.minimum(ridx_v[ch, gsl], vm - 1)
            c16 = jnp.minimum(cidx_v[ch, gsl], vm - 1)
            rb16 = r16 + (r16 >> 7) * 896
            cb16 = c16 + (c16 >> 7) * 896
            for e in range(embed):
                a_e = (e // 8) * tile_row_words + (e % 8) * 128
                ribuf[e, gsl] = rb16 + a_e
                cibuf[e, gsl] = cb16 + a_e
            return carry

        lax.fori_loop(0, ngrp, build, 0)

        for e in range(embed):
            pltpu.make_async_copy(dflat_r.at[ribuf.at[e]], re_v.at[e], sem).start()
            pltpu.make_async_copy(dflat_c.at[cibuf.at[e]], ce_v.at[e], sem).start()

        pltpu.make_async_copy(row_bias.at[ridx_v.at[ch]],
                              rb_v.at[pl.ds(off, _CH)], sem).wait()
        pltpu.make_async_copy(col_bias.at[cidx_v.at[ch]],
                              cb_v.at[pl.ds(off, _CH)], sem).wait()
        for e in range(embed):
            pltpu.make_async_copy(dflat_r.at[ribuf.at[e]], re_v.at[e], sem).wait()
            pltpu.make_async_copy(dflat_c.at[cibuf.at[e]], ce_v.at[e], sem).wait()

        if vt:
            def fix(g, carry):
                gsl = pl.ds(g * _LANES, _LANES)
                r16 = ridx_v[ch, gsl]
                c16 = cidx_v[ch, gsl]
                rmask = r16 >= vm
                cmask = c16 >= vm
                nhit = plsc.all_reduce_population_count(rmask | cmask)

                @pl.when(nhit[0] > 0)
                def _():
                    rt16 = jnp.clip(r16 - vm, 0, vt - 1)
                    ct16 = jnp.clip(c16 - vm, 0, vt - 1)
                    for e in range(embed):
                        tv = plsc.load_gather(tr_v, [rt16 + e * vt])
                        re_v[e, gsl] = jnp.where(rmask, tv, re_v[e, gsl])
                        cv = plsc.load_gather(tc_v, [ct16 + e * vt])
                        ce_v[e, gsl] = jnp.where(cmask, cv, ce_v[e, gsl])
                return carry

            lax.fori_loop(0, ngrp, fix, 0)

        def comp(g, carry):
            gsl = pl.ds(g * _LANES, _LANES)
            osl = pl.ds(off + g * _LANES, _LANES)
            acc = rb_v[osl] + cb_v[osl] + gb
            for e in range(embed):
                acc = acc + re_v[e, gsl] * ce_v[e, gsl]
            out_v[osl] = acc
            return carry

        lax.fori_loop(0, ngrp, comp, 0)

    pltpu.sync_copy(out_v, out.at[pl.ds(base, nper)])


def _sc_matfac(rid, cid, dflat_r, dflat_c, tail_r, tail_c, rb, cb, gvec,
               embed, vm, vt):
    batch = rid.shape[0]
    nper = batch // _NW
    assert nper * _NW == batch and nper % _CH == 0

    run = pl.kernel(
        functools.partial(_sc_body, nper, embed, vm, vt),
        out_type=jax.ShapeDtypeStruct((batch,), jnp.float32),
        mesh=plsc.VectorSubcoreMesh(core_axis_name="c", subcore_axis_name="s"),
        compiler_params=pltpu.CompilerParams(needs_layout_passes=False),
        scratch_types=[
            pltpu.VMEM((nper // _CH, _CH), jnp.int32),
            pltpu.VMEM((nper // _CH, _CH), jnp.int32),
            pltpu.VMEM((embed, _CH), jnp.int32),
            pltpu.VMEM((embed, _CH), jnp.int32),
            pltpu.VMEM((embed, _CH), jnp.float32),
            pltpu.VMEM((embed, _CH), jnp.float32),
            pltpu.VMEM((max(vt, 1) * embed,), jnp.float32),
            pltpu.VMEM((max(vt, 1) * embed,), jnp.float32),
            pltpu.VMEM((nper,), jnp.float32),
            pltpu.VMEM((nper,), jnp.float32),
            pltpu.VMEM((_LANES,), jnp.float32),
            pltpu.VMEM((nper,), jnp.float32),
            pltpu.SemaphoreType.DMA,
        ],
    )
    return run(rid, cid, dflat_r, dflat_c, tail_r, tail_c, rb, cb, gvec)


def _flat_native_view(table, vm):
    embed = table.shape[1]
    main = table[:vm].T
    d4 = main.reshape(embed // 8, 8, vm // 128, 128)
    return d4.transpose(0, 2, 1, 3).reshape(-1)


def kernel(row_id, col_id, row_emb, row_bias, col_emb, col_bias, global_bias):
    batch = row_id.shape[0]
    vocab, embed = row_emb.shape
    vm = (vocab // 128) * 128
    vt = vocab - vm
    rid = row_id.astype(jnp.int32)
    cid = col_id.astype(jnp.int32)
    dflat_r = _flat_native_view(row_emb, vm)
    dflat_c = _flat_native_view(col_emb, vm)
    if vt:
        tail_r = row_emb[vm:].T.reshape(-1)
        tail_c = col_emb[vm:].T.reshape(-1)
    else:
        tail_r = jnp.zeros((embed,), jnp.float32)
        tail_c = jnp.zeros((embed,), jnp.float32)
    rb = row_bias.reshape(-1)
    cb = col_bias.reshape(-1)
    gvec = jnp.full((_LANES,), global_bias, jnp.float32)
    out = _sc_matfac(rid, cid, dflat_r, dflat_c, tail_r, tail_c, rb, cb, gvec,
                     embed, vm, vt)
    return out.reshape(batch, 1)


# speedup vs baseline: 2.3867x; 1.0946x over previous
import functools

import jax
import jax.numpy as jnp
from jax import lax
from jax.experimental import pallas as pl
from jax.experimental.pallas import tpu as pltpu
from jax.experimental.pallas import tpu_sc as plsc

_NC = 2
_NS = 16
_NW = _NC * _NS
_LANES = 16
_CH = 128


def _sc_body(nper, embed, vm, vt,
             row_id, col_id, dflat_r, dflat_c, tail_r, tail_c,
             rb_flat, cb_flat, tail_rb, tail_cb, gvec, out,
             ridx_v, cidx_v, ribuf, cibuf, re_v, ce_v, tr_v, tc_v,
             trb_v, tcb_v, rb_v, cb_v, gb_v, out_v, sems):
    wid = lax.axis_index("s") * _NC + lax.axis_index("c")
    base = wid * nper
    nchunk = nper // _CH
    ngrp = _CH // _LANES
    tile_row_words = (vm // 128) * 1024

    for j in range(nchunk):
        pltpu.sync_copy(row_id.at[pl.ds(base + j * _CH, _CH)], ridx_v.at[j])
        pltpu.sync_copy(col_id.at[pl.ds(base + j * _CH, _CH)], cidx_v.at[j])
    if vt:
        pltpu.sync_copy(tail_r, tr_v)
        pltpu.sync_copy(tail_c, tc_v)
        pltpu.sync_copy(tail_rb, trb_v)
        pltpu.sync_copy(tail_cb, tcb_v)
    pltpu.sync_copy(gvec, gb_v)
    gb = gb_v[...]

    def build(ch, p):
        rib = ribuf.at[p]
        cib = cibuf.at[p]

        def body(g, carry):
            gsl = pl.ds(g * _LANES, _LANES)
            r16 = jnp.minimum(ridx_v[ch, gsl], vm - 1)
            c16 = jnp.minimum(cidx_v[ch, gsl], vm - 1)
            rb16 = r16 + (r16 >> 7) * 896
            cb16 = c16 + (c16 >> 7) * 896
            rib[embed, gsl] = r16
            cib[embed, gsl] = c16
            for e in range(embed):
                a_e = (e // 8) * tile_row_words + (e % 8) * 128
                rib[e, gsl] = rb16 + a_e
                cib[e, gsl] = cb16 + a_e
            return carry

        lax.fori_loop(0, ngrp, body, 0)

    def transfers(ch, p):
        off = ch * _CH
        yield pltpu.make_async_copy(rb_flat.at[ribuf.at[p].at[embed]],
                                    rb_v.at[pl.ds(off, _CH)], sems[p])
        yield pltpu.make_async_copy(cb_flat.at[cibuf.at[p].at[embed]],
                                    cb_v.at[pl.ds(off, _CH)], sems[p])
        for e in range(embed):
            yield pltpu.make_async_copy(dflat_r.at[ribuf.at[p].at[e]],
                                        re_v.at[p].at[e], sems[p])
            yield pltpu.make_async_copy(dflat_c.at[cibuf.at[p].at[e]],
                                        ce_v.at[p].at[e], sems[p])

    def fire(ch, p):
        for c in transfers(ch, p):
            c.start()

    def drain(ch, p):
        for c in transfers(ch, p):
            c.wait()

    def fix(ch, p):
        if not vt:
            return

        def body(g, carry):
            gsl = pl.ds(g * _LANES, _LANES)
            osl = pl.ds(ch * _CH + g * _LANES, _LANES)
            r16 = ridx_v[ch, gsl]
            c16 = cidx_v[ch, gsl]
            rmask = r16 >= vm
            cmask = c16 >= vm
            nhit = plsc.all_reduce_population_count(rmask | cmask)

            @pl.when(nhit[0] > 0)
            def _():
                rt16 = jnp.clip(r16 - vm, 0, vt - 1)
                ct16 = jnp.clip(c16 - vm, 0, vt - 1)
                rb_v[osl] = jnp.where(rmask, plsc.load_gather(trb_v, [rt16]),
                                      rb_v[osl])
                cb_v[osl] = jnp.where(cmask, plsc.load_gather(tcb_v, [ct16]),
                                      cb_v[osl])
                for e in range(embed):
                    tv = plsc.load_gather(tr_v, [rt16 + e * vt])
                    re_v[p, e, gsl] = jnp.where(rmask, tv, re_v[p, e, gsl])
                    cv = plsc.load_gather(tc_v, [ct16 + e * vt])
                    ce_v[p, e, gsl] = jnp.where(cmask, cv, ce_v[p, e, gsl])
            return carry

        lax.fori_loop(0, ngrp, body, 0)

    def comp(ch, p):
        def body(g, carry):
            gsl = pl.ds(g * _LANES, _LANES)
            osl = pl.ds(ch * _CH + g * _LANES, _LANES)
            acc = rb_v[osl] + cb_v[osl] + gb
            for e in range(embed):
                acc = acc + re_v[p, e, gsl] * ce_v[p, e, gsl]
            out_v[osl] = acc
            return carry

        lax.fori_loop(0, ngrp, body, 0)

    build(0, 0)
    fire(0, 0)
    for ch in range(nchunk):
        p = ch & 1
        if ch + 1 < nchunk:
            build(ch + 1, 1 - p)
            fire(ch + 1, 1 - p)
        drain(ch, p)
        fix(ch, p)
        comp(ch, p)

    pltpu.sync_copy(out_v, out.at[pl.ds(base, nper)])


def _sc_matfac(rid, cid, dflat_r, dflat_c, tail_r, tail_c,
               rb_flat, cb_flat, tail_rb, tail_cb, gvec, embed, vm, vt):
    batch = rid.shape[0]
    nper = batch // _NW
    assert nper * _NW == batch and nper % _CH == 0
    vts = max(vt, 1)

    run = pl.kernel(
        functools.partial(_sc_body, nper, embed, vm, vt),
        out_type=jax.ShapeDtypeStruct((batch,), jnp.float32),
        mesh=plsc.VectorSubcoreMesh(core_axis_name="c", subcore_axis_name="s"),
        compiler_params=pltpu.CompilerParams(needs_layout_passes=False),
        scratch_types=[
            pltpu.VMEM((nper // _CH, _CH), jnp.int32),
            pltpu.VMEM((nper // _CH, _CH), jnp.int32),
            pltpu.VMEM((2, embed + 1, _CH), jnp.int32),
            pltpu.VMEM((2, embed + 1, _CH), jnp.int32),
            pltpu.VMEM((2, embed, _CH), jnp.float32),
            pltpu.VMEM((2, embed, _CH), jnp.float32),
            pltpu.VMEM((vts * embed,), jnp.float32),
            pltpu.VMEM((vts * embed,), jnp.float32),
            pltpu.VMEM((vts,), jnp.float32),
            pltpu.VMEM((vts,), jnp.float32),
            pltpu.VMEM((nper,), jnp.float32),
            pltpu.VMEM((nper,), jnp.float32),
            pltpu.VMEM((_LANES,), jnp.float32),
            pltpu.VMEM((nper,), jnp.float32),
            [pltpu.SemaphoreType.DMA, pltpu.SemaphoreType.DMA],
        ],
    )
    return run(rid, cid, dflat_r, dflat_c, tail_r, tail_c,
               rb_flat, cb_flat, tail_rb, tail_cb, gvec)


def _flat_native_view(table, vm):
    embed = table.shape[1]
    main = table[:vm].T
    d4 = main.reshape(embed // 8, 8, vm // 128, 128)
    return d4.transpose(0, 2, 1, 3).reshape(-1)


def kernel(row_id, col_id, row_emb, row_bias, col_emb, col_bias, global_bias):
    batch = row_id.shape[0]
    vocab, embed = row_emb.shape
    vm = (vocab // 128) * 128
    vt = vocab - vm
    rid = row_id.astype(jnp.int32)
    cid = col_id.astype(jnp.int32)
    dflat_r = _flat_native_view(row_emb, vm)
    dflat_c = _flat_native_view(col_emb, vm)
    rb_flat = row_bias[:vm].T.reshape(-1)
    cb_flat = col_bias[:vm].T.reshape(-1)
    if vt:
        tail_r = row_emb[vm:].T.reshape(-1)
        tail_c = col_emb[vm:].T.reshape(-1)
        tail_rb = row_bias[vm:].T.reshape(-1)
        tail_cb = col_bias[vm:].T.reshape(-1)
    else:
        tail_r = jnp.zeros((embed,), jnp.float32)
        tail_c = jnp.zeros((embed,), jnp.float32)
        tail_rb = jnp.zeros((1,), jnp.float32)
        tail_cb = jnp.zeros((1,), jnp.float32)
    gvec = jnp.full((_LANES,), global_bias, jnp.float32)
    out = _sc_matfac(rid, cid, dflat_r, dflat_c, tail_r, tail_c,
                     rb_flat, cb_flat, tail_rb, tail_cb, gvec,
                     embed, vm, vt)
    return out.reshape(batch, 1)


# speedup vs baseline: 2.3917x; 1.0021x over previous
import functools

import jax
import jax.numpy as jnp
from jax import lax
from jax.experimental import pallas as pl
from jax.experimental.pallas import tpu as pltpu
from jax.experimental.pallas import tpu_sc as plsc

_NC = 2
_NS = 16
_NW = _NC * _NS
_LANES = 16
_CH = 128


def _sc_body(nper, embed, vm, vt,
             row_id, col_id, dflat_r, dflat_c, tail_r, tail_c,
             rb_flat, cb_flat, tail_rb, tail_cb, gvec, out,
             ridx_v, cidx_v, ribuf, cibuf, re_v, ce_v, tr_v, tc_v,
             trb_v, tcb_v, rb_v, cb_v, gb_v, out_v, sems):
    wid = lax.axis_index("s") * _NC + lax.axis_index("c")
    base = wid * nper
    nchunk = nper // _CH
    ngrp = _CH // _LANES
    tile_row_words = (vm // 128) * 1024

    for j in range(nchunk):
        pltpu.sync_copy(row_id.at[pl.ds(base + j * _CH, _CH)], ridx_v.at[j])
        pltpu.sync_copy(col_id.at[pl.ds(base + j * _CH, _CH)], cidx_v.at[j])
    if vt:
        pltpu.sync_copy(tail_r, tr_v)
        pltpu.sync_copy(tail_c, tc_v)
        pltpu.sync_copy(tail_rb, trb_v)
        pltpu.sync_copy(tail_cb, tcb_v)
    pltpu.sync_copy(gvec, gb_v)
    gb = gb_v[...]

    def build(ch, p):
        rib = ribuf.at[p]
        cib = cibuf.at[p]

        def body(g, carry):
            gsl = pl.ds(g * _LANES, _LANES)
            r16 = jnp.minimum(ridx_v[ch, gsl], vm - 1)
            c16 = jnp.minimum(cidx_v[ch, gsl], vm - 1)
            rb16 = r16 + (r16 >> 7) * 896
            cb16 = c16 + (c16 >> 7) * 896
            rib[embed, gsl] = r16
            cib[embed, gsl] = c16
            for e in range(embed):
                a_e = (e // 8) * tile_row_words + (e % 8) * 128
                rib[e, gsl] = rb16 + a_e
                cib[e, gsl] = cb16 + a_e
            return carry

        lax.fori_loop(0, ngrp, body, 0)

    def transfers(ch, p):
        off = ch * _CH
        yield pltpu.make_async_copy(rb_flat.at[ribuf.at[p].at[embed]],
                                    rb_v.at[pl.ds(off, _CH)], sems[p])
        yield pltpu.make_async_copy(cb_flat.at[cibuf.at[p].at[embed]],
                                    cb_v.at[pl.ds(off, _CH)], sems[p])
        for e in range(embed):
            yield pltpu.make_async_copy(dflat_r.at[ribuf.at[p].at[e]],
                                        re_v.at[p].at[e], sems[p])
            yield pltpu.make_async_copy(dflat_c.at[cibuf.at[p].at[e]],
                                        ce_v.at[p].at[e], sems[p])

    def fire(ch, p):
        for c in transfers(ch, p):
            c.start()

    def drain(ch, p):
        for c in transfers(ch, p):
            c.wait()

    def fix(ch, p):
        if not vt:
            return

        def body(g, carry):
            gsl = pl.ds(g * _LANES, _LANES)
            osl = pl.ds(ch * _CH + g * _LANES, _LANES)
            r16 = ridx_v[ch, gsl]
            c16 = cidx_v[ch, gsl]
            rmask = r16 >= vm
            cmask = c16 >= vm
            nhit = plsc.all_reduce_population_count(rmask | cmask)

            @pl.when(nhit[0] > 0)
            def _():
                rt16 = jnp.clip(r16 - vm, 0, vt - 1)
                ct16 = jnp.clip(c16 - vm, 0, vt - 1)
                rb_v[osl] = jnp.where(rmask, plsc.load_gather(trb_v, [rt16]),
                                      rb_v[osl])
                cb_v[osl] = jnp.where(cmask, plsc.load_gather(tcb_v, [ct16]),
                                      cb_v[osl])
                for e in range(embed):
                    tv = plsc.load_gather(tr_v, [rt16 + e * vt])
                    re_v[p, e, gsl] = jnp.where(rmask, tv, re_v[p, e, gsl])
                    cv = plsc.load_gather(tc_v, [ct16 + e * vt])
                    ce_v[p, e, gsl] = jnp.where(cmask, cv, ce_v[p, e, gsl])
            return carry

        lax.fori_loop(0, ngrp, body, 0)

    def comp(ch, p):
        def body(g, carry):
            gsl = pl.ds(g * _LANES, _LANES)
            osl = pl.ds(ch * _CH + g * _LANES, _LANES)
            acc = rb_v[osl] + cb_v[osl] + gb
            for e in range(embed):
                acc = acc + re_v[p, e, gsl] * ce_v[p, e, gsl]
            out_v[osl] = acc
            return carry

        lax.fori_loop(0, ngrp, body, 0)

    build(0, 0)
    fire(0, 0)
    for ch in range(nchunk):
        p = ch & 1
        if ch + 1 < nchunk:
            build(ch + 1, 1 - p)
            fire(ch + 1, 1 - p)
        drain(ch, p)
        fix(ch, p)
        comp(ch, p)

    pltpu.sync_copy(out_v, out.at[pl.ds(base, nper)])


def _sc_matfac(rid, cid, dflat_r, dflat_c, tail_r, tail_c,
               rb_flat, cb_flat, tail_rb, tail_cb, gvec, embed, vm, vt):
    batch = rid.shape[0]
    nper = batch // _NW
    assert nper * _NW == batch and nper % _CH == 0
    vts = max(vt, 1)

    run = pl.kernel(
        functools.partial(_sc_body, nper, embed, vm, vt),
        out_type=jax.ShapeDtypeStruct((batch,), jnp.float32),
        mesh=plsc.VectorSubcoreMesh(core_axis_name="c", subcore_axis_name="s"),
        compiler_params=pltpu.CompilerParams(needs_layout_passes=False),
        scratch_types=[
            pltpu.VMEM((nper // _CH, _CH), jnp.int32),
            pltpu.VMEM((nper // _CH, _CH), jnp.int32),
            pltpu.VMEM((2, embed + 1, _CH), jnp.int32),
            pltpu.VMEM((2, embed + 1, _CH), jnp.int32),
            pltpu.VMEM((2, embed, _CH), jnp.float32),
            pltpu.VMEM((2, embed, _CH), jnp.float32),
            pltpu.VMEM((vts * embed,), jnp.float32),
            pltpu.VMEM((vts * embed,), jnp.float32),
            pltpu.VMEM((vts,), jnp.float32),
            pltpu.VMEM((vts,), jnp.float32),
            pltpu.VMEM((nper,), jnp.float32),
            pltpu.VMEM((nper,), jnp.float32),
            pltpu.VMEM((_LANES,), jnp.float32),
            pltpu.VMEM((nper,), jnp.float32),
            [pltpu.SemaphoreType.DMA, pltpu.SemaphoreType.DMA],
        ],
    )
    return run(rid, cid, dflat_r, dflat_c, tail_r, tail_c,
               rb_flat, cb_flat, tail_rb, tail_cb, gvec)


def _flat_native_view(table, vm):
    embed = table.shape[1]
    main = table[:vm].T
    d4 = main.reshape(embed // 8, 8, vm // 128, 128)
    return d4.transpose(0, 2, 1, 3).reshape(-1)


def kernel(row_id, col_id, row_emb, row_bias, col_emb, col_bias, global_bias):
    batch = row_id.shape[0]
    vocab, embed = row_emb.shape
    vm = (vocab // 128) * 128
    vt = vocab - vm
    rid = row_id.astype(jnp.int32)
    cid = col_id.astype(jnp.int32)
    dflat_r = _flat_native_view(row_emb, vm)
    dflat_c = _flat_native_view(col_emb, vm)
    rb_flat = row_bias[:vm].reshape(-1)
    cb_flat = col_bias[:vm].reshape(-1)
    if vt:
        tail_r = row_emb[vm:].T.reshape(-1)
        tail_c = col_emb[vm:].T.reshape(-1)
        tail_rb = row_bias[vm:].T.reshape(-1)
        tail_cb = col_bias[vm:].T.reshape(-1)
    else:
        tail_r = jnp.zeros((embed,), jnp.float32)
        tail_c = jnp.zeros((embed,), jnp.float32)
        tail_rb = jnp.zeros((1,), jnp.float32)
        tail_cb = jnp.zeros((1,), jnp.float32)
    gvec = jnp.full((_LANES,), global_bias, jnp.float32)
    out = _sc_matfac(rid, cid, dflat_r, dflat_c, tail_r, tail_c,
                     rb_flat, cb_flat, tail_rb, tail_cb, gvec,
                     embed, vm, vt)
    return out.reshape(batch, 1)


# speedup vs baseline: 2.5291x; 1.0575x over previous
import functools

import jax
import jax.numpy as jnp
from jax import lax
from jax.experimental import pallas as pl
from jax.experimental.pallas import tpu as pltpu
from jax.experimental.pallas import tpu_sc as plsc

_NC = 2
_NS = 16
_NW = _NC * _NS
_LANES = 16
_CH = 128


def _build_idx(idx_ref, ch, ibuf, embed, vm, tile_row_words, with_bias_row):
    ngrp = _CH // _LANES

    def body(g, carry):
        gsl = pl.ds(g * _LANES, _LANES)
        r16 = jnp.minimum(idx_ref[ch, gsl], vm - 1)
        rb16 = r16 + (r16 >> 7) * 896
        if with_bias_row:
            ibuf[embed, gsl] = r16
        for e in range(embed):
            a_e = (e // 8) * tile_row_words + (e % 8) * 128
            ibuf[e, gsl] = rb16 + a_e
        return carry

    lax.fori_loop(0, ngrp, body, 0)


def _fix_tail(idx_ref, ch, p, emb_v, tail_v, bias_v, tbias_v, embed, vm, vt):
    ngrp = _CH // _LANES

    def body(g, carry):
        gsl = pl.ds(g * _LANES, _LANES)
        osl = pl.ds(ch * _CH + g * _LANES, _LANES)
        r16 = idx_ref[ch, gsl]
        rmask = r16 >= vm
        nhit = plsc.all_reduce_population_count(rmask)

        @pl.when(nhit[0] > 0)
        def _():
            rt16 = jnp.clip(r16 - vm, 0, vt - 1)
            if bias_v is not None:
                bias_v[osl] = jnp.where(rmask, plsc.load_gather(tbias_v, [rt16]),
                                        bias_v[osl])
            if emb_v is not None:
                for e in range(embed):
                    tv = plsc.load_gather(tail_v, [rt16 + e * vt])
                    emb_v[p, e, gsl] = jnp.where(rmask, tv, emb_v[p, e, gsl])
        return carry

    lax.fori_loop(0, ngrp, body, 0)


def _row_body(nper, embed, vm, vt,
              row_id, dflat_r, tail_r, out_fm,
              ridx_v, ribuf, re_v, tr_v, sems, sem_st):
    wid = lax.axis_index("s") * _NC + lax.axis_index("c")
    base = wid * nper
    nchunk = nper // _CH
    tile_row_words = (vm // 128) * 1024

    for j in range(nchunk):
        pltpu.sync_copy(row_id.at[pl.ds(base + j * _CH, _CH)], ridx_v.at[j])
    if vt:
        pltpu.sync_copy(tail_r, tr_v)

    def transfers(p):
        for e in range(embed):
            yield pltpu.make_async_copy(dflat_r.at[ribuf.at[p].at[e]],
                                        re_v.at[p].at[e], sems[p])

    def stores(ch, p):
        for e in range(embed):
            yield pltpu.make_async_copy(
                re_v.at[p].at[e],
                out_fm.at[e, pl.ds(base + ch * _CH, _CH)], sem_st)

    _build_idx(ridx_v, 0, ribuf.at[0], embed, vm, tile_row_words, False)
    for c in transfers(0):
        c.start()
    for ch in range(nchunk):
        p = ch & 1
        if ch + 1 < nchunk:
            _build_idx(ridx_v, ch + 1, ribuf.at[1 - p], embed, vm,
                       tile_row_words, False)
            for c in transfers(1 - p):
                c.start()
        for c in transfers(p):
            c.wait()
        if vt:
            _fix_tail(ridx_v, ch, p, re_v, tr_v, None, None, embed, vm, vt)
        for c in stores(ch, p):
            c.start()
        if ch >= 1:
            for c in stores(ch - 1, 1 - p):
                c.wait()
    for c in stores(nchunk - 1, (nchunk - 1) & 1):
        c.wait()


def _col_body(nper, embed, vm, vt,
              row_id, col_id, dflat_c, tail_c, rb_flat, cb_flat,
              tail_rb, tail_cb, re_fm, gvec, out,
              ridx_v, cidx_v, cibuf, ce_v, ra_v, tc_v, trb_v, tcb_v,
              rb_v, cb_v, gb_v, out_v, sems):
    wid = lax.axis_index("s") * _NC + lax.axis_index("c")
    base = wid * nper
    nchunk = nper // _CH
    ngrp = _CH // _LANES
    tile_row_words = (vm // 128) * 1024

    for j in range(nchunk):
        pltpu.sync_copy(row_id.at[pl.ds(base + j * _CH, _CH)], ridx_v.at[j])
        pltpu.sync_copy(col_id.at[pl.ds(base + j * _CH, _CH)], cidx_v.at[j])
    if vt:
        pltpu.sync_copy(tail_c, tc_v)
        pltpu.sync_copy(tail_rb, trb_v)
        pltpu.sync_copy(tail_cb, tcb_v)
    pltpu.sync_copy(gvec, gb_v)
    gb = gb_v[...]

    def transfers(ch, p):
        off = ch * _CH
        yield pltpu.make_async_copy(rb_flat.at[cibuf.at[p].at[embed]],
                                    rb_v.at[pl.ds(off, _CH)], sems[p])
        yield pltpu.make_async_copy(cb_flat.at[cibuf.at[p].at[embed + 1]],
                                    cb_v.at[pl.ds(off, _CH)], sems[p])
        for e in range(embed):
            yield pltpu.make_async_copy(dflat_c.at[cibuf.at[p].at[e]],
                                        ce_v.at[p].at[e], sems[p])
            yield pltpu.make_async_copy(
                re_fm.at[e, pl.ds(base + off, _CH)], ra_v.at[p].at[e], sems[p])

    def build(ch, p):
        cib = cibuf.at[p]

        def body(g, carry):
            gsl = pl.ds(g * _LANES, _LANES)
            c16 = jnp.minimum(cidx_v[ch, gsl], vm - 1)
            cb16 = c16 + (c16 >> 7) * 896
            cib[embed, gsl] = jnp.minimum(ridx_v[ch, gsl], vm - 1)
            cib[embed + 1, gsl] = c16
            for e in range(embed):
                a_e = (e // 8) * tile_row_words + (e % 8) * 128
                cib[e, gsl] = cb16 + a_e
            return carry

        lax.fori_loop(0, ngrp, body, 0)

    def comp(ch, p):
        def body(g, carry):
            gsl = pl.ds(g * _LANES, _LANES)
            osl = pl.ds(ch * _CH + g * _LANES, _LANES)
            acc = rb_v[osl] + cb_v[osl] + gb
            for e in range(embed):
                acc = acc + ra_v[p, e, gsl] * ce_v[p, e, gsl]
            out_v[osl] = acc
            return carry

        lax.fori_loop(0, ngrp, body, 0)

    build(0, 0)
    for c in transfers(0, 0):
        c.start()
    for ch in range(nchunk):
        p = ch & 1
        if ch + 1 < nchunk:
            build(ch + 1, 1 - p)
            for c in transfers(ch + 1, 1 - p):
                c.start()
        for c in transfers(ch, p):
            c.wait()
        if vt:
            _fix_tail(cidx_v, ch, p, ce_v, tc_v, cb_v, tcb_v, embed, vm, vt)
            _fix_tail(ridx_v, ch, p, None, None, rb_v, trb_v, embed, vm, vt)
        comp(ch, p)

    pltpu.sync_copy(out_v, out.at[pl.ds(base, nper)])


def _mk_common(nper, embed, vts):
    return [
        pltpu.VMEM((nper // _CH, _CH), jnp.int32),
    ]


def _sc_matfac(rid, cid, dflat_r, dflat_c, tail_r, tail_c,
               rb_flat, cb_flat, tail_rb, tail_cb, gvec, embed, vm, vt):
    batch = rid.shape[0]
    nper = batch // _NW
    assert nper * _NW == batch and nper % _CH == 0
    vts = max(vt, 1)
    mesh = plsc.VectorSubcoreMesh(core_axis_name="c", subcore_axis_name="s")
    params = pltpu.CompilerParams(needs_layout_passes=False)

    run_a = pl.kernel(
        functools.partial(_row_body, nper, embed, vm, vt),
        out_type=jax.ShapeDtypeStruct((embed, batch), jnp.float32),
        mesh=mesh,
        compiler_params=params,
        scratch_types=[
            pltpu.VMEM((nper // _CH, _CH), jnp.int32),
            pltpu.VMEM((2, embed, _CH), jnp.int32),
            pltpu.VMEM((2, embed, _CH), jnp.float32),
            pltpu.VMEM((vts * embed,), jnp.float32),
            [pltpu.SemaphoreType.DMA, pltpu.SemaphoreType.DMA],
            pltpu.SemaphoreType.DMA,
        ],
    )
    re_fm = run_a(rid, dflat_r, tail_r)

    run_b = pl.kernel(
        functools.partial(_col_body, nper, embed, vm, vt),
        out_type=jax.ShapeDtypeStruct((batch,), jnp.float32),
        mesh=mesh,
        compiler_params=params,
        scratch_types=[
            pltpu.VMEM((nper // _CH, _CH), jnp.int32),
            pltpu.VMEM((nper // _CH, _CH), jnp.int32),
            pltpu.VMEM((2, embed + 2, _CH), jnp.int32),
            pltpu.VMEM((2, embed, _CH), jnp.float32),
            pltpu.VMEM((2, embed, _CH), jnp.float32),
            pltpu.VMEM((vts * embed,), jnp.float32),
            pltpu.VMEM((vts,), jnp.float32),
            pltpu.VMEM((vts,), jnp.float32),
            pltpu.VMEM((nper,), jnp.float32),
            pltpu.VMEM((nper,), jnp.float32),
            pltpu.VMEM((_LANES,), jnp.float32),
            pltpu.VMEM((nper,), jnp.float32),
            [pltpu.SemaphoreType.DMA, pltpu.SemaphoreType.DMA],
        ],
    )
    return run_b(rid, cid, dflat_c, tail_c, rb_flat, cb_flat, tail_rb, tail_cb,
                 re_fm, gvec)


def _flat_native_view(table, vm):
    embed = table.shape[1]
    main = table[:vm].T
    d4 = main.reshape(embed // 8, 8, vm // 128, 128)
    return d4.transpose(0, 2, 1, 3).reshape(-1)


def kernel(row_id, col_id, row_emb, row_bias, col_emb, col_bias, global_bias):
    batch = row_id.shape[0]
    vocab, embed = row_emb.shape
    vm = (vocab // 128) * 128
    vt = vocab - vm
    rid = row_id.astype(jnp.int32)
    cid = col_id.astype(jnp.int32)
    dflat_r = _flat_native_view(row_emb, vm)
    dflat_c = _flat_native_view(col_emb, vm)
    rb_flat = row_bias[:vm].reshape(-1)
    cb_flat = col_bias[:vm].reshape(-1)
    if vt:
        tail_r = row_emb[vm:].T.reshape(-1)
        tail_c = col_emb[vm:].T.reshape(-1)
        tail_rb = row_bias[vm:].T.reshape(-1)
        tail_cb = col_bias[vm:].T.reshape(-1)
    else:
        tail_r = jnp.zeros((embed,), jnp.float32)
        tail_c = jnp.zeros((embed,), jnp.float32)
        tail_rb = jnp.zeros((1,), jnp.float32)
        tail_cb = jnp.zeros((1,), jnp.float32)
    gvec = jnp.full((_LANES,), global_bias, jnp.float32)
    out = _sc_matfac(rid, cid, dflat_r, dflat_c, tail_r, tail_c,
                     rb_flat, cb_flat, tail_rb, tail_cb, gvec,
                     embed, vm, vt)
    return out.reshape(batch, 1)
